# baseline (speedup 1.0000x reference)
import functools

import jax
import jax.numpy as jnp
from jax import lax
from jax.experimental import pallas as pl
from jax.experimental.pallas import tpu as pltpu
from jax.experimental.pallas import tpu_sc as plsc

VOCAB = 1_000_000
DIM = 64
B = 16384
C = 20
K = 5

NC = 2
NS = 16
NW = NC * NS
BW = B // NW
CH = 32
NCHUNK = BW // CH
UC = CH * C
NEGC = CH * K
U_SPLIT = 128
NEG_SPLIT = 80

_BLKC = 8192
_TR_GRID = (VOCAB + 2 * _BLKC - 1) // (2 * _BLKC)
_VROWS2 = _TR_GRID * _BLKC
_VROWS = 2 * _VROWS2


def _tc_detranspose(t):
  def body(t1_ref, t2_ref, o_ref):
    e_top = jnp.concatenate(
        [jnp.eye(DIM, dtype=jnp.float32),
         jnp.zeros((DIM, DIM), jnp.float32)], axis=1)
    e_bot = jnp.concatenate(
        [jnp.zeros((DIM, DIM), jnp.float32),
         jnp.eye(DIM, dtype=jnp.float32)], axis=1)
    dn = (((0,), (0,)), ((), ()))
    o_ref[...] = (
        lax.dot_general(t1_ref[...], e_top, dn,
                        preferred_element_type=jnp.float32)
        + lax.dot_general(t2_ref[...], e_bot, dn,
                          preferred_element_type=jnp.float32))

  return pl.pallas_call(
      body,
      grid=(_TR_GRID,),
      in_specs=[
          pl.BlockSpec((DIM, _BLKC), lambda i: (0, 2 * i)),
          pl.BlockSpec((DIM, _BLKC),
                       lambda i: (0, jnp.minimum(2 * i + 1, VOCAB // _BLKC))),
      ],
      out_specs=pl.BlockSpec((_BLKC, 2 * DIM), lambda i: (i, 0)),
      out_shape=jax.ShapeDtypeStruct((_VROWS2, 2 * DIM), jnp.float32),
  )(t, t)


def _sc_pool_u(pos_u_f, u_rm):
  mesh = plsc.VectorSubcoreMesh(core_axis_name="c", subcore_axis_name="s")

  @functools.partial(
      pl.kernel,
      mesh=mesh,
      compiler_params=pltpu.CompilerParams(use_tc_tiling_on_sc=False),
      out_type=jax.ShapeDtypeStruct((B, DIM), jnp.float32),
      scratch_types=[
          pltpu.VMEM((BW * C,), jnp.int32),
          pltpu.VMEM((UC, DIM), jnp.float32),
          pltpu.VMEM((UC, DIM), jnp.float32),
          pltpu.VMEM((CH, DIM), jnp.float32),
          pltpu.SemaphoreType.DMA,
          pltpu.SemaphoreType.DMA,
      ],
  )
  def k(pos_u_hbm, u_hbm, emb_u_hbm, uidx, urows0, urows1, uacc,
        sem0, sem1):
    wid = lax.axis_index("s") * NC + lax.axis_index("c")
    base = wid * BW
    urows = (urows0, urows1)
    sems = (sem0, sem1)

    pltpu.sync_copy(pos_u_hbm.at[pl.ds(base * C, BW * C)], uidx)

    def copy_descs(t, b, make):
      return [make(
          u_hbm.at[uidx.at[pl.ds(t * UC + j * U_SPLIT, U_SPLIT)]],
          urows[b].at[pl.ds(j * U_SPLIT, U_SPLIT), :], sems[b])
          for j in range(UC // U_SPLIT)]

    copy_descs(0, 0, pltpu.async_copy)

    def pair_body(tp, carry):
      for b in range(2):
        t = tp * 2 + b

        @pl.when(t + 1 < NCHUNK)
        def _():
          copy_descs(t + 1, 1 - b, pltpu.async_copy)

        for d in copy_descs(t, b, pltpu.make_async_copy):
          d.wait()

        def item_body(i, c2):
          def c_body(c, accs):
            r = i * C + c
            return tuple(accs[g] + urows[b][r, pl.ds(g * 16, 16)]
                         for g in range(DIM // 16))
          accs = lax.fori_loop(
              0, C, c_body,
              tuple(jnp.zeros((16,), jnp.float32)
                    for _ in range(DIM // 16)))
          for g in range(DIM // 16):
            uacc[i, pl.ds(g * 16, 16)] = accs[g] * (1.0 / C)
          return c2

        lax.fori_loop(0, CH, item_body, 0)
        pltpu.sync_copy(uacc, emb_u_hbm.at[pl.ds(base + t * CH, CH), :])
      return carry

    lax.fori_loop(0, NCHUNK // 2, pair_body, 0)

  return k(pos_u_f, u_rm)


_VCH = 64
_VNCHUNK = BW // _VCH


def _sc_gather_v(pos_v_f, neg_v_f, v_rm):
  mesh = plsc.VectorSubcoreMesh(core_axis_name="c", subcore_axis_name="s")

  @functools.partial(
      pl.kernel,
      mesh=mesh,
      compiler_params=pltpu.CompilerParams(use_tc_tiling_on_sc=False),
      out_type=[
          jax.ShapeDtypeStruct((B, DIM), jnp.float32),
          jax.ShapeDtypeStruct((B * K, DIM), jnp.float32),
      ],
      scratch_types=[
          pltpu.VMEM((BW * K,), jnp.int32),
          pltpu.VMEM((BW,), jnp.int32),
          pltpu.VMEM((K * _VCH, DIM), jnp.float32),
          pltpu.VMEM((K * _VCH, DIM), jnp.float32),
          pltpu.VMEM((_VCH, DIM), jnp.float32),
          pltpu.VMEM((_VCH, DIM), jnp.float32),
          pltpu.SemaphoreType.DMA,
          pltpu.SemaphoreType.DMA,
      ],
  )
  def k(pos_v_hbm, neg_v_hbm, v_hbm, emb_v_hbm, emb_neg_hbm,
        nidx, vidx, nrows0, nrows1, vrows0, vrows1, sem0, sem1):
    wid = lax.axis_index("s") * NC + lax.axis_index("c")
    base = wid * BW
    nrows = (nrows0, nrows1)
    vrows = (vrows0, vrows1)
    sems = (sem0, sem1)

    for kk in range(K):
      pltpu.sync_copy(neg_v_hbm.at[pl.ds(kk * B + base, BW)],
                      nidx.at[pl.ds(kk * BW, BW)])
    pltpu.sync_copy(pos_v_hbm.at[pl.ds(base, BW)], vidx)

    def copy_descs(t, b, make):
      ds = [make(v_hbm.at[nidx.at[pl.ds(kk * BW + t * _VCH, _VCH)]],
                 nrows[b].at[pl.ds(kk * _VCH, _VCH), :], sems[b])
            for kk in range(K)]
      ds.append(make(v_hbm.at[vidx.at[pl.ds(t * _VCH, _VCH)]],
                     vrows[b], sems[b]))
      return ds

    copy_descs(0, 0, pltpu.async_copy)

    def pair_body(tp, carry):
      for b in range(2):
        t = tp * 2 + b

        @pl.when(t + 1 < _VNCHUNK)
        def _():
          copy_descs(t + 1, 1 - b, pltpu.async_copy)

        for d in copy_descs(t, b, pltpu.make_async_copy):
          d.wait()

        b0 = base + t * _VCH
        pltpu.sync_copy(vrows[b], emb_v_hbm.at[pl.ds(b0, _VCH), :])
        for kk in range(K):
          pltpu.sync_copy(nrows[b].at[pl.ds(kk * _VCH, _VCH), :],
                          emb_neg_hbm.at[pl.ds(kk * B + b0, _VCH), :])
      return carry

    lax.fori_loop(0, _VNCHUNK // 2, pair_body, 0)

  return k(pos_v_f, neg_v_f, v_rm)


_TC_BLK = 4096


def _tc_loss(emb_u2, emb_v2, emb_neg3):
  grid = B // _TC_BLK
  hb = _TC_BLK // 2

  def body(u_ref, v_ref, n_ref, out_ref):
    i = pl.program_id(0)

    def terms(prod, sign):
      s_e = jnp.sum(prod[:, :DIM], axis=1)
      s_o = jnp.sum(prod[:, DIM:], axis=1)
      tot = jnp.float32(0.0)
      for s in (s_e, s_o):
        s = jnp.clip(sign * s, -10.0, 10.0)
        tot = tot + jnp.sum(jnp.log1p(jnp.exp(-s)))
      return tot

    u = u_ref[...]
    part = terms(u * v_ref[...], 1.0)
    for kk in range(K):
      part = part + terms(u * n_ref[kk], -1.0)
    part = part * (1.0 / B)

    @pl.when(i == 0)
    def _():
      out_ref[...] = jnp.zeros((1, 1), jnp.float32)

    out_ref[...] += jnp.full((1, 1), part, jnp.float32)

  out = pl.pallas_call(
      body,
      grid=(grid,),
      in_specs=[
          pl.BlockSpec((hb, 2 * DIM), lambda i: (i, 0)),
          pl.BlockSpec((hb, 2 * DIM), lambda i: (i, 0)),
          pl.BlockSpec((K, hb, 2 * DIM), lambda i: (0, i, 0)),
      ],
      out_specs=pl.BlockSpec((1, 1), lambda i: (0, 0)),
      out_shape=jax.ShapeDtypeStruct((1, 1), jnp.float32),
  )(emb_u2, emb_v2, emb_neg3)
  return out[0, 0]


def _perm(v):
  b = v >> 13
  j = v & (_BLKC - 1)
  return ((b >> 1) << 14) + (j << 1) + (b & 1)


def kernel(pos_u, pos_v, neg_v, U_weight, V_weight):
  pos_u_f = _perm(pos_u.astype(jnp.int32).reshape(B * C))
  neg_v_f = _perm(neg_v.astype(jnp.int32).T.reshape(K * B))
  pos_v_f = _perm(pos_v.astype(jnp.int32).reshape(B))
  u2 = _tc_detranspose(U_weight.T)
  emb_u = _sc_pool_u(pos_u_f, u2.reshape(_VROWS, DIM))
  v2 = _tc_detranspose(V_weight.T)
  emb_v, emb_neg = _sc_gather_v(pos_v_f, neg_v_f, v2.reshape(_VROWS, DIM))
  return _tc_loss(emb_u.reshape(B // 2, 2 * DIM),
                  emb_v.reshape(B // 2, 2 * DIM),
                  emb_neg.reshape(K, B // 2, 2 * DIM))


# speedup vs baseline: 1.0502x; 1.0502x over previous
import functools

import jax
import jax.numpy as jnp
from jax import lax
from jax.experimental import pallas as pl
from jax.experimental.pallas import tpu as pltpu
from jax.experimental.pallas import tpu_sc as plsc

VOCAB = 1_000_000
DIM = 64
B = 16384
C = 20
K = 5

NC = 2
NS = 16
NW = NC * NS
BW = B // NW
CH = 32
NCHUNK = BW // CH
UC = CH * C
NEGC = CH * K
U_SPLIT = 128
NEG_SPLIT = 80

_BLKC = 8192
_TR_GRID = (VOCAB + 2 * _BLKC - 1) // (2 * _BLKC)
_VROWS2 = _TR_GRID * _BLKC
_VROWS = 2 * _VROWS2


def _tc_detranspose(ut, vt):
  def body(u1_ref, u2_ref, v1_ref, v2_ref, u_ref, v_ref):
    e_top = jnp.concatenate(
        [jnp.eye(DIM, dtype=jnp.float32),
         jnp.zeros((DIM, DIM), jnp.float32)], axis=1)
    e_bot = jnp.concatenate(
        [jnp.zeros((DIM, DIM), jnp.float32),
         jnp.eye(DIM, dtype=jnp.float32)], axis=1)
    dn = (((0,), (0,)), ((), ()))
    u_ref[...] = (
        lax.dot_general(u1_ref[...], e_top, dn,
                        preferred_element_type=jnp.float32)
        + lax.dot_general(u2_ref[...], e_bot, dn,
                          preferred_element_type=jnp.float32))
    v_ref[...] = (
        lax.dot_general(v1_ref[...], e_top, dn,
                        preferred_element_type=jnp.float32)
        + lax.dot_general(v2_ref[...], e_bot, dn,
                          preferred_element_type=jnp.float32))

  return pl.pallas_call(
      body,
      grid=(_TR_GRID,),
      in_specs=[
          pl.BlockSpec((DIM, _BLKC), lambda i: (0, 2 * i)),
          pl.BlockSpec((DIM, _BLKC),
                       lambda i: (0, jnp.minimum(2 * i + 1, VOCAB // _BLKC))),
          pl.BlockSpec((DIM, _BLKC), lambda i: (0, 2 * i)),
          pl.BlockSpec((DIM, _BLKC),
                       lambda i: (0, jnp.minimum(2 * i + 1, VOCAB // _BLKC))),
      ],
      out_specs=[
          pl.BlockSpec((_BLKC, 2 * DIM), lambda i: (i, 0)),
          pl.BlockSpec((_BLKC, 2 * DIM), lambda i: (i, 0)),
      ],
      out_shape=[
          jax.ShapeDtypeStruct((_VROWS2, 2 * DIM), jnp.float32),
          jax.ShapeDtypeStruct((_VROWS2, 2 * DIM), jnp.float32),
      ],
  )(ut, ut, vt, vt)


def _sc_gather_pool(pos_u_f, pos_v_f, neg_v_f, u_rm, v_rm):
  mesh = plsc.VectorSubcoreMesh(core_axis_name="c", subcore_axis_name="s")

  @functools.partial(
      pl.kernel,
      mesh=mesh,
      compiler_params=pltpu.CompilerParams(use_tc_tiling_on_sc=False),
      out_type=[
          jax.ShapeDtypeStruct((B, DIM), jnp.float32),
          jax.ShapeDtypeStruct((B, DIM), jnp.float32),
          jax.ShapeDtypeStruct((B * K, DIM), jnp.float32),
      ],
      scratch_types=[
          pltpu.VMEM((BW * C,), jnp.int32),
          pltpu.VMEM((BW * K,), jnp.int32),
          pltpu.VMEM((BW,), jnp.int32),
          pltpu.VMEM((UC, DIM), jnp.float32),
          pltpu.VMEM((UC, DIM), jnp.float32),
          pltpu.VMEM((NEGC, DIM), jnp.float32),
          pltpu.VMEM((NEGC, DIM), jnp.float32),
          pltpu.VMEM((CH, DIM), jnp.float32),
          pltpu.VMEM((CH, DIM), jnp.float32),
          pltpu.VMEM((CH, DIM), jnp.float32),
          pltpu.SemaphoreType.DMA,
          pltpu.SemaphoreType.DMA,
      ],
  )
  def k(pos_u_hbm, pos_v_hbm, neg_v_hbm, u_hbm, v_hbm,
        emb_u_hbm, emb_v_hbm, emb_neg_hbm,
        uidx, nidx, vidx, urows0, urows1, nrows0, nrows1, vrows0, vrows1,
        uacc, sem0, sem1):
    wid = lax.axis_index("s") * NC + lax.axis_index("c")
    base = wid * BW
    urows = (urows0, urows1)
    nrows = (nrows0, nrows1)
    vrows = (vrows0, vrows1)
    sems = (sem0, sem1)

    pltpu.sync_copy(pos_u_hbm.at[pl.ds(base * C, BW * C)], uidx)
    for kk in range(K):
      pltpu.sync_copy(neg_v_hbm.at[pl.ds(kk * B + base, BW)],
                      nidx.at[pl.ds(kk * BW, BW)])
    pltpu.sync_copy(pos_v_hbm.at[pl.ds(base, BW)], vidx)

    def copy_descs(t, b, make):
      ds = []
      for j in range(UC // U_SPLIT):
        ds.append(make(
            u_hbm.at[uidx.at[pl.ds(t * UC + j * U_SPLIT, U_SPLIT)]],
            urows[b].at[pl.ds(j * U_SPLIT, U_SPLIT), :], sems[b]))
      for kk in range(K):
        ds.append(make(
            v_hbm.at[nidx.at[pl.ds(kk * BW + t * CH, CH)]],
            nrows[b].at[pl.ds(kk * CH, CH), :], sems[b]))
      ds.append(make(v_hbm.at[vidx.at[pl.ds(t * CH, CH)]],
                     vrows[b], sems[b]))
      return ds

    copy_descs(0, 0, pltpu.async_copy)

    def pair_body(tp, carry):
      for b in range(2):
        t = tp * 2 + b
        @pl.when(t + 1 < NCHUNK)
        def _():
          copy_descs(t + 1, 1 - b, pltpu.async_copy)

        for d in copy_descs(t, b, pltpu.make_async_copy):
          d.wait()

        def item_body(i, c2):
          def c_body(c, accs):
            r = i * C + c
            return tuple(accs[g] + urows[b][r, pl.ds(g * 16, 16)]
                         for g in range(DIM // 16))
          accs = lax.fori_loop(
              0, C, c_body,
              tuple(jnp.zeros((16,), jnp.float32)
                    for _ in range(DIM // 16)))
          for g in range(DIM // 16):
            uacc[i, pl.ds(g * 16, 16)] = accs[g] * (1.0 / C)
          return c2

        lax.fori_loop(0, CH, item_body, 0)

        b0 = base + t * CH
        pltpu.sync_copy(uacc, emb_u_hbm.at[pl.ds(b0, CH), :])
        pltpu.sync_copy(vrows[b], emb_v_hbm.at[pl.ds(b0, CH), :])
        for kk in range(K):
          pltpu.sync_copy(nrows[b].at[pl.ds(kk * CH, CH), :],
                          emb_neg_hbm.at[pl.ds(kk * B + b0, CH), :])
      return carry

    lax.fori_loop(0, NCHUNK // 2, pair_body, 0)

  return k(pos_u_f, pos_v_f, neg_v_f, u_rm, v_rm)


_TC_BLK = 4096


def _tc_loss(emb_u2, emb_v2, emb_neg3):
  grid = B // _TC_BLK
  hb = _TC_BLK // 2

  def body(u_ref, v_ref, n_ref, out_ref):
    i = pl.program_id(0)

    def terms(prod, sign):
      s_e = jnp.sum(prod[:, :DIM], axis=1)
      s_o = jnp.sum(prod[:, DIM:], axis=1)
      tot = jnp.float32(0.0)
      for s in (s_e, s_o):
        s = jnp.clip(sign * s, -10.0, 10.0)
        tot = tot + jnp.sum(jnp.log1p(jnp.exp(-s)))
      return tot

    u = u_ref[...]
    part = terms(u * v_ref[...], 1.0)
    for kk in range(K):
      part = part + terms(u * n_ref[kk], -1.0)
    part = part * (1.0 / B)

    @pl.when(i == 0)
    def _():
      out_ref[...] = jnp.zeros((1, 1), jnp.float32)

    out_ref[...] += jnp.full((1, 1), part, jnp.float32)

  out = pl.pallas_call(
      body,
      grid=(grid,),
      in_specs=[
          pl.BlockSpec((hb, 2 * DIM), lambda i: (i, 0)),
          pl.BlockSpec((hb, 2 * DIM), lambda i: (i, 0)),
          pl.BlockSpec((K, hb, 2 * DIM), lambda i: (0, i, 0)),
      ],
      out_specs=pl.BlockSpec((1, 1), lambda i: (0, 0)),
      out_shape=jax.ShapeDtypeStruct((1, 1), jnp.float32),
  )(emb_u2, emb_v2, emb_neg3)
  return out[0, 0]


def _perm(v):
  b = v >> 13
  j = v & (_BLKC - 1)
  return ((b >> 1) << 14) + (j << 1) + (b & 1)


def kernel(pos_u, pos_v, neg_v, U_weight, V_weight):
  pos_u_f = _perm(pos_u.astype(jnp.int32).reshape(B * C))
  neg_v_f = _perm(neg_v.astype(jnp.int32).T.reshape(K * B))
  pos_v_f = _perm(pos_v.astype(jnp.int32).reshape(B))
  u2, v2 = _tc_detranspose(U_weight.T, V_weight.T)
  u_rm = u2.reshape(_VROWS, DIM)
  v_rm = v2.reshape(_VROWS, DIM)
  emb_u, emb_v, emb_neg = _sc_gather_pool(
      pos_u_f, pos_v_f, neg_v_f, u_rm, v_rm)
  return _tc_loss(emb_u.reshape(B // 2, 2 * DIM),
                  emb_v.reshape(B // 2, 2 * DIM),
                  emb_neg.reshape(K, B // 2, 2 * DIM))
